# baseline (speedup 1.0000x reference)
import jax
import jax.numpy as jnp
from jax.experimental import pallas as pl
from jax.experimental.pallas import tpu as pltpu

B = 16
Q = 5000
C = 92
F = 4 * Q
THRESH = 0.7

P = 4
NSTEP = C // P


def _body(x_ref, sc_ref, lb_ref, m91r, mAr, sr, lblr):
    i = pl.program_id(0)
    xs = [x_ref[k] for k in range(P)]
    cmax = jnp.maximum(jnp.maximum(xs[0], xs[1]), jnp.maximum(xs[2], xs[3]))

    @pl.when(i == 0)
    def _():
        m = xs[0]
        lbl = jnp.zeros((B, Q), jnp.int32)
        for k in range(1, P):
            gt = xs[k] > m
            m = jnp.where(gt, xs[k], m)
            lbl = jnp.where(gt, k, lbl)
        m91r[...] = m
        lblr[...] = lbl
        mAr[...] = cmax
        acc = jnp.exp(xs[0] - cmax)
        for k in range(1, P):
            acc = acc + jnp.exp(xs[k] - cmax)
        sr[...] = acc

    @pl.when((i > 0) & (i < NSTEP - 1))
    def _():
        m = m91r[...]
        lbl = lblr[...]
        for k in range(P):
            gt = xs[k] > m
            m = jnp.where(gt, xs[k], m)
            lbl = jnp.where(gt, P * i + k, lbl)
        m91r[...] = m
        lblr[...] = lbl
        mo = mAr[...]
        mn = jnp.maximum(mo, cmax)
        acc = sr[...] * jnp.exp(mo - mn)
        for k in range(P):
            acc = acc + jnp.exp(xs[k] - mn)
        sr[...] = acc
        mAr[...] = mn

    @pl.when(i == NSTEP - 1)
    def _():
        m = m91r[...]
        lbl = lblr[...]
        for k in range(P - 1):
            gt = xs[k] > m
            m = jnp.where(gt, xs[k], m)
            lbl = jnp.where(gt, P * (NSTEP - 1) + k, lbl)
        mo = mAr[...]
        mn = jnp.maximum(mo, cmax)
        acc = sr[...] * jnp.exp(mo - mn)
        for k in range(P):
            acc = acc + jnp.exp(xs[k] - mn)
        score = jnp.exp(m - mn) / acc
        keep = score > THRESH
        sc_ref[...] = jnp.where(keep, score, 0.0)
        lb_ref[...] = jnp.where(keep, lbl, 0)


_scores = pl.pallas_call(
    _body,
    grid=(NSTEP,),
    in_specs=[
        pl.BlockSpec((P, B, Q), lambda i: (i, 0, 0)),
    ],
    out_specs=[
        pl.BlockSpec((B, Q), lambda i: (0, 0)),
        pl.BlockSpec((B, Q), lambda i: (0, 0)),
    ],
    out_shape=[
        jax.ShapeDtypeStruct((B, Q), jnp.float32),
        jax.ShapeDtypeStruct((B, Q), jnp.int32),
    ],
    scratch_shapes=[
        pltpu.VMEM((B, Q), jnp.float32),
        pltpu.VMEM((B, Q), jnp.float32),
        pltpu.VMEM((B, Q), jnp.float32),
        pltpu.VMEM((B, Q), jnp.int32),
    ],
)


def _box_body(bx_ref, s4_ref, sc_ref, bo_ref):
    keep4 = sc_ref[...] > THRESH
    bo_ref[...] = jnp.where(keep4, bx_ref[...] * s4_ref[...], 0.0)


_boxes = pl.pallas_call(
    _box_body,
    out_shape=jax.ShapeDtypeStruct((B, F), jnp.float32),
)


@jax.jit
def kernel(pred_logits, pred_boxes, target_sizes):
    lgT = jnp.transpose(pred_logits, (2, 0, 1))
    ts = target_sizes.astype(jnp.float32)
    img_h = ts[:, 0]
    img_w = ts[:, 1]
    scale = jnp.stack([img_w, img_h, img_w, img_h], axis=1)
    s4 = jnp.broadcast_to(scale[:, None, :], (B, Q, 4)).reshape(B, F)
    bx = pred_boxes.reshape(B, F)
    scores2d, labels2d = _scores(lgT)
    sc4 = jnp.broadcast_to(scores2d[:, :, None], (B, Q, 4)).reshape(B, F)
    boxes_f = _boxes(bx, s4, sc4)
    keep = scores2d > THRESH
    return scores2d, labels2d, boxes_f.reshape(B, Q, 4), keep


# speedup vs baseline: 1.4250x; 1.4250x over previous
import jax
import jax.numpy as jnp
from jax.experimental import pallas as pl
from jax.experimental.pallas import tpu as pltpu

B = 16
Q = 5000
C = 92
F = 4 * Q
THRESH = 0.7

P = 4
NSTEP = C // P


def _body(x_ref, sc_ref, lb_ref, m91r, mAr, sr, lblr):
    i = pl.program_id(0)
    xs = [x_ref[k] for k in range(P)]
    cmax = jnp.maximum(jnp.maximum(xs[0], xs[1]), jnp.maximum(xs[2], xs[3]))

    @pl.when(i == 0)
    def _():
        m = xs[0]
        lbl = jnp.zeros((B, Q), jnp.int32)
        for k in range(1, P):
            gt = xs[k] > m
            m = jnp.where(gt, xs[k], m)
            lbl = jnp.where(gt, k, lbl)
        m91r[...] = m
        lblr[...] = lbl
        mAr[...] = cmax
        acc = jnp.exp(xs[0] - cmax)
        for k in range(1, P):
            acc = acc + jnp.exp(xs[k] - cmax)
        sr[...] = acc

    @pl.when((i > 0) & (i < NSTEP - 1))
    def _():
        m = m91r[...]
        lbl = lblr[...]
        for k in range(P):
            gt = xs[k] > m
            m = jnp.where(gt, xs[k], m)
            lbl = jnp.where(gt, P * i + k, lbl)
        m91r[...] = m
        lblr[...] = lbl
        mo = mAr[...]
        mn = jnp.maximum(mo, cmax)
        acc = sr[...] * jnp.exp(mo - mn)
        for k in range(P):
            acc = acc + jnp.exp(xs[k] - mn)
        sr[...] = acc
        mAr[...] = mn

    @pl.when(i == NSTEP - 1)
    def _():
        m = m91r[...]
        lbl = lblr[...]
        for k in range(P - 1):
            gt = xs[k] > m
            m = jnp.where(gt, xs[k], m)
            lbl = jnp.where(gt, P * (NSTEP - 1) + k, lbl)
        mo = mAr[...]
        mn = jnp.maximum(mo, cmax)
        acc = sr[...] * jnp.exp(mo - mn)
        for k in range(P):
            acc = acc + jnp.exp(xs[k] - mn)
        score = jnp.exp(m - mn) / acc
        keep = score > THRESH
        sc_ref[...] = jnp.where(keep, score, 0.0)
        lb_ref[...] = jnp.where(keep, lbl, 0)


_scores = pl.pallas_call(
    _body,
    grid=(NSTEP,),
    in_specs=[
        pl.BlockSpec((P, B, Q), lambda i: (i, 0, 0)),
    ],
    out_specs=[
        pl.BlockSpec((B, Q), lambda i: (0, 0)),
        pl.BlockSpec((B, Q), lambda i: (0, 0)),
    ],
    out_shape=[
        jax.ShapeDtypeStruct((B, Q), jnp.float32),
        jax.ShapeDtypeStruct((B, Q), jnp.int32),
    ],
    scratch_shapes=[
        pltpu.VMEM((B, Q), jnp.float32),
        pltpu.VMEM((B, Q), jnp.float32),
        pltpu.VMEM((B, Q), jnp.float32),
        pltpu.VMEM((B, Q), jnp.int32),
    ],
)


def _box_body(bx_ref, s4_ref, sc_ref, bo_ref):
    kf = jnp.where(sc_ref[...] > THRESH, 1.0, 0.0)
    k4 = pltpu.repeat(kf, 4, axis=1)
    bo_ref[...] = bx_ref[...] * s4_ref[...] * k4


_boxes = pl.pallas_call(
    _box_body,
    out_shape=jax.ShapeDtypeStruct((B, F), jnp.float32),
)


@jax.jit
def kernel(pred_logits, pred_boxes, target_sizes):
    lgT = jnp.transpose(pred_logits, (2, 0, 1))
    ts = target_sizes.astype(jnp.float32)
    img_h = ts[:, 0]
    img_w = ts[:, 1]
    s4 = jnp.where((jnp.arange(F)[None, :] % 2) == 0,
                   img_w[:, None], img_h[:, None])
    bx = pred_boxes.reshape(B, F)
    scores2d, labels2d = _scores(lgT)
    boxes_f = _boxes(bx, s4, scores2d)
    keep = scores2d > THRESH
    return scores2d, labels2d, boxes_f.reshape(B, Q, 4), keep


# speedup vs baseline: 2.4281x; 1.7039x over previous
import jax
import jax.numpy as jnp
from jax.experimental import pallas as pl
from jax.experimental.pallas import tpu as pltpu

B = 16
Q = 5000
C = 92
F = 4 * Q
THRESH = 0.7

P = 4
NSTEP = C // P


def _body(x_ref, sc_ref, lb_ref, m91r, mAr, sr, lblr):
    i = pl.program_id(0)
    xs = [x_ref[k] for k in range(P)]
    cmax = jnp.maximum(jnp.maximum(xs[0], xs[1]), jnp.maximum(xs[2], xs[3]))

    @pl.when(i == 0)
    def _():
        m = xs[0]
        lbl = jnp.zeros((B, Q), jnp.int32)
        for k in range(1, P):
            gt = xs[k] > m
            m = jnp.where(gt, xs[k], m)
            lbl = jnp.where(gt, k, lbl)
        m91r[...] = m
        lblr[...] = lbl
        mAr[...] = cmax
        acc = jnp.exp(xs[0] - cmax)
        for k in range(1, P):
            acc = acc + jnp.exp(xs[k] - cmax)
        sr[...] = acc

    @pl.when((i > 0) & (i < NSTEP - 1))
    def _():
        m = m91r[...]
        lbl = lblr[...]
        for k in range(P):
            gt = xs[k] > m
            m = jnp.where(gt, xs[k], m)
            lbl = jnp.where(gt, P * i + k, lbl)
        m91r[...] = m
        lblr[...] = lbl
        mo = mAr[...]
        mn = jnp.maximum(mo, cmax)
        acc = sr[...] * jnp.exp(mo - mn)
        for k in range(P):
            acc = acc + jnp.exp(xs[k] - mn)
        sr[...] = acc
        mAr[...] = mn

    @pl.when(i == NSTEP - 1)
    def _():
        m = m91r[...]
        lbl = lblr[...]
        for k in range(P - 1):
            gt = xs[k] > m
            m = jnp.where(gt, xs[k], m)
            lbl = jnp.where(gt, P * (NSTEP - 1) + k, lbl)
        mo = mAr[...]
        mn = jnp.maximum(mo, cmax)
        acc = sr[...] * jnp.exp(mo - mn)
        for k in range(P):
            acc = acc + jnp.exp(xs[k] - mn)
        score = jnp.exp(m - mn) / acc
        keep = score > THRESH
        sc_ref[...] = jnp.where(keep, score, 0.0)
        lb_ref[...] = jnp.where(keep, lbl, 0)


_scores = pl.pallas_call(
    _body,
    grid=(NSTEP,),
    in_specs=[
        pl.BlockSpec((P, B, Q), lambda i: (i, 0, 0)),
    ],
    out_specs=[
        pl.BlockSpec((B, Q), lambda i: (0, 0)),
        pl.BlockSpec((B, Q), lambda i: (0, 0)),
    ],
    out_shape=[
        jax.ShapeDtypeStruct((B, Q), jnp.float32),
        jax.ShapeDtypeStruct((B, Q), jnp.int32),
    ],
    scratch_shapes=[
        pltpu.VMEM((B, Q), jnp.float32),
        pltpu.VMEM((B, Q), jnp.float32),
        pltpu.VMEM((B, Q), jnp.float32),
        pltpu.VMEM((B, Q), jnp.int32),
    ],
)


def _box_body(bx_ref, scl_ref, sc_ref, bo_ref):
    kp = sc_ref[...] > THRESH
    for c in range(4):
        bo_ref[c] = jnp.where(kp, bx_ref[c] * scl_ref[c], 0.0)


_boxes = pl.pallas_call(
    _box_body,
    out_shape=jax.ShapeDtypeStruct((4, B, Q), jnp.float32),
)


@jax.jit
def kernel(pred_logits, pred_boxes, target_sizes):
    lgT = jnp.transpose(pred_logits, (2, 0, 1))
    ts = target_sizes.astype(jnp.float32)
    img_h = ts[:, 0]
    img_w = ts[:, 1]
    scl = jnp.stack([img_w, img_h, img_w, img_h], axis=0)[:, :, None]
    bxt = jnp.transpose(pred_boxes, (2, 0, 1))
    scores2d, labels2d = _scores(lgT)
    boxes_t = _boxes(bxt, scl, scores2d)
    boxes = jnp.transpose(boxes_t, (1, 2, 0))
    keep = scores2d > THRESH
    return scores2d, labels2d, boxes, keep
